# baseline (speedup 1.0000x reference)
import functools

import jax
import jax.numpy as jnp
from jax import lax
from jax.experimental import pallas as pl
from jax.experimental.pallas import tpu as pltpu
from jax.experimental.pallas import tpu_sc as plsc

B = 16384
D = 64
DP = 2 * D
E_ROWS = 100000
PREP_W = 1024
N_PREP = 98
ENT2_ROWS = N_PREP * (PREP_W // 2)
REL2_ROWS = N_PREP * PREP_W
NC, NS = 2, 16
NW = NC * NS
PER_W = B // NW
CHUNK = 256
IDX_CHUNK = 128
TC_BLK = 2048
NB = B // TC_BLK


def _mxu_t(x):
    eye = jax.lax.broadcasted_iota(jnp.int32, (D, D), 0) == \
        jax.lax.broadcasted_iota(jnp.int32, (D, D), 1)
    return jax.lax.dot_general(
        x, eye.astype(jnp.float32), (((0,), (0,)), ((), ())),
        precision=jax.lax.Precision.HIGHEST,
        preferred_element_type=jnp.float32)


def _ent_prep_body(et_r, out_r):
    t = _mxu_t(et_r[...])
    out_r[:, :D] = t[: PREP_W // 2]
    out_r[:, D:] = t[PREP_W // 2:]


def _rel_prep_body(rnt_r, rht_r, out_r):
    out_r[:, :D] = _mxu_t(rnt_r[...])
    out_r[:, D:] = _mxu_t(rht_r[...])


_PREP_PARAMS = pltpu.CompilerParams(dimension_semantics=("parallel",))


def _ent_prep(et):
    return pl.pallas_call(
        _ent_prep_body,
        grid=(N_PREP,),
        in_specs=[pl.BlockSpec((D, PREP_W), lambda i: (0, i))],
        out_specs=pl.BlockSpec((PREP_W // 2, DP), lambda i: (i, 0)),
        out_shape=jax.ShapeDtypeStruct((ENT2_ROWS, DP), jnp.float32),
        compiler_params=_PREP_PARAMS,
    )(et)


def _rel_prep(rnt, rht):
    return pl.pallas_call(
        _rel_prep_body,
        grid=(N_PREP,),
        in_specs=[pl.BlockSpec((D, PREP_W), lambda i: (0, i))] * 2,
        out_specs=pl.BlockSpec((PREP_W, DP), lambda i: (i, 0)),
        out_shape=jax.ShapeDtypeStruct((REL2_ROWS, DP), jnp.float32),
        compiler_params=_PREP_PARAMS,
    )(rnt, rht)


def _make_sc_gather(n_sets, table_rows):
    mesh = plsc.VectorSubcoreMesh(core_axis_name="c", subcore_axis_name="s")
    row_t = jax.ShapeDtypeStruct((B, DP), jnp.float32)
    n_items = n_sets * (PER_W // CHUNK)

    @functools.partial(
        pl.kernel,
        mesh=mesh,
        out_type=[row_t] * n_sets,
        scratch_types=[
            pltpu.VMEM((n_sets * PER_W,), jnp.int32),
            pltpu.VMEM((CHUNK, DP), jnp.float32),
            pltpu.VMEM((CHUNK, DP), jnp.float32),
            pltpu.SemaphoreType.DMA,
            pltpu.SemaphoreType.DMA,
            pltpu.SemaphoreType.DMA,
        ],
        compiler_params=pltpu.CompilerParams(use_tc_tiling_on_sc=True),
    )
    def k(*refs):
        idx_hbms = refs[:n_sets]
        tab_hbm = refs[n_sets]
        outs = refs[n_sets + 1:2 * n_sets + 1]
        idx_v, buf0, buf1, gsem0, gsem1, ssem = refs[2 * n_sets + 1:]
        wid = lax.axis_index("s") * NC + lax.axis_index("c")
        base = wid * PER_W
        bufs = (buf0, buf1)
        gsems = (gsem0, gsem1)

        icopies = [
            pltpu.async_copy(src.at[pl.ds(base, PER_W)],
                             idx_v.at[pl.ds(s * PER_W, PER_W)], ssem)
            for s, src in enumerate(idx_hbms)
        ]
        for cp in icopies:
            cp.wait()

        def fire(item, buf, gsem):
            st, chunk = divmod(item, PER_W // CHUNK)
            cps = []
            for c in range(CHUNK // IDX_CHUNK):
                off = st * PER_W + chunk * CHUNK + c * IDX_CHUNK
                cps.append(pltpu.async_copy(
                    tab_hbm.at[idx_v.at[pl.ds(off, IDX_CHUNK)]],
                    buf.at[pl.ds(c * IDX_CHUNK, IDX_CHUNK)],
                    gsem,
                ))
            return cps

        def store(item, buf):
            st, chunk = divmod(item, PER_W // CHUNK)
            return pltpu.async_copy(
                buf, outs[st].at[pl.ds(base + chunk * CHUNK, CHUNK)], ssem)

        gathers = [None] * n_items
        stores = [None] * n_items
        gathers[0] = fire(0, bufs[0], gsems[0])
        for item in range(n_items):
            par = item % 2
            for cp in gathers[item]:
                cp.wait()
            if item + 1 < n_items:
                if item >= 1:
                    stores[item - 1].wait()
                gathers[item + 1] = fire(item + 1, bufs[1 - par],
                                         gsems[1 - par])
            stores[item] = store(item, bufs[par])
        stores[n_items - 1].wait()
        if n_items >= 2:
            stores[n_items - 2].wait()

    return k


def _half(pair_block, sel_col):
    return jnp.where(sel_col == 0.0, pair_block[:, :D], pair_block[:, D:])


def _groupsum(parts):
    k = len(parts)
    x = jnp.concatenate(parts, axis=1)
    g = jax.lax.broadcasted_iota(jnp.int32, (k * D, k), 0) // D == \
        jax.lax.broadcasted_iota(jnp.int32, (k * D, k), 1)
    return jax.lax.dot_general(
        x, g.astype(jnp.float32), (((1,), (0,)), ((), ())),
        precision=jax.lax.Precision.HIGHEST,
        preferred_element_type=jnp.float32)


def _tc_loss_body(hp_r, tp_r, chp_r, ctp_r, rel_r, sel_r, out_r):
    sel = sel_r[...]
    hd = _half(hp_r[...], sel[:, 0:1])
    tl = _half(tp_r[...], sel[:, 1:2])
    c_h = _half(chp_r[...], sel[:, 2:3])
    c_t = _half(ctp_r[...], sel[:, 3:4])
    rel = rel_r[...]
    rn = rel[:, :D]
    rh = rel[:, D:]

    d = hd - tl
    dc = c_h - c_t
    s = _groupsum([rn * d, rn * dc])
    pv = d - s[:, 0:1] * rn + rh + 1e-6
    nv = dc - s[:, 1:2] * rn + rh + 1e-6
    q = _groupsum([pv * pv, nv * nv, hd * hd, tl * tl, c_h * c_h, c_t * c_t])
    pos = jnp.sqrt(q[:, 0:1])
    neg = jnp.sqrt(q[:, 1:2])
    total = jnp.sum(jnp.maximum(pos - neg + 1.0, 0.0))
    nrm = jnp.sqrt(q[:, 2:6])
    total += jnp.sum(jnp.maximum(nrm - 1.0, 0.0))
    out_r[0, 0, 0] = total


def _tc_loss(headp, tailp, chp, ctp, rel, sels):
    out = pl.pallas_call(
        _tc_loss_body,
        grid=(NB,),
        in_specs=[pl.BlockSpec((TC_BLK, DP), lambda i: (i, 0))] * 5
        + [pl.BlockSpec((TC_BLK, 4), lambda i: (i, 0))],
        out_specs=pl.BlockSpec((1, 1, 1), lambda i: (i, 0, 0),
                               memory_space=pltpu.SMEM),
        out_shape=jax.ShapeDtypeStruct((NB, 1, 1), jnp.float32),
        compiler_params=pltpu.CompilerParams(
            dimension_semantics=("parallel",)),
    )(headp, tailp, chp, ctp, rel, sels)
    return jnp.sum(out)


def kernel(current_triples, corrupted_triples, entity_embedding,
           relation_norm_embedding, relation_hyper_embedding):
    h = current_triples[:, 0]
    t = current_triples[:, 1]
    r = current_triples[:, 2]
    h_c = corrupted_triples[:, 0]
    t_c = corrupted_triples[:, 1]

    ent2 = _ent_prep(entity_embedding.T)
    rel2 = _rel_prep(relation_norm_embedding.T, relation_hyper_embedding.T)

    def fold(e):
        return ((e >> 10) << 9) + (e & 511)

    gather_ent = _make_sc_gather(4, ENT2_ROWS)
    gather_rel = _make_sc_gather(1, REL2_ROWS)
    headp, tailp, chp, ctp = gather_ent(
        fold(h), fold(t), fold(h_c), fold(t_c), ent2)
    (rel,) = gather_rel(r, rel2)

    sels = jnp.stack(
        [(h >> 9) & 1, (t >> 9) & 1, (h_c >> 9) & 1, (t_c >> 9) & 1],
        axis=1).astype(jnp.float32)
    return _tc_loss(headp, tailp, chp, ctp, rel, sels)


# speedup vs baseline: 1.5210x; 1.5210x over previous
import functools

import jax
import jax.numpy as jnp
from jax import lax
from jax.experimental import pallas as pl
from jax.experimental.pallas import tpu as pltpu
from jax.experimental.pallas import tpu_sc as plsc

B = 16384
D = 64
DP = 2 * D
E_ROWS = 100000
PREP_W = 1024
N_PREP = 98
ENT2_ROWS = N_PREP * (PREP_W // 2)
REL2_ROWS = N_PREP * PREP_W
NC, NS = 2, 16
NW = NC * NS
PER_W = B // NW
CHUNK = 256
IDX_CHUNK = 128
TC_BLK = 2048
NB = B // TC_BLK


def _mxu_t(x):
    eye = jax.lax.broadcasted_iota(jnp.int32, (D, D), 0) == \
        jax.lax.broadcasted_iota(jnp.int32, (D, D), 1)
    return jax.lax.dot_general(
        x, eye.astype(jnp.float32), (((0,), (0,)), ((), ())),
        preferred_element_type=jnp.float32)


def _prep_body(et_r, rnt_r, rht_r, ent_o, rel_o):
    t = _mxu_t(et_r[...])
    ent_o[:, :D] = t[: PREP_W // 2]
    ent_o[:, D:] = t[PREP_W // 2:]
    rel_o[:, :D] = _mxu_t(rnt_r[...])
    rel_o[:, D:] = _mxu_t(rht_r[...])


def _prep(et, rnt, rht):
    return pl.pallas_call(
        _prep_body,
        grid=(N_PREP,),
        in_specs=[pl.BlockSpec((D, PREP_W), lambda i: (0, i))] * 3,
        out_specs=[pl.BlockSpec((PREP_W // 2, DP), lambda i: (i, 0)),
                   pl.BlockSpec((PREP_W, DP), lambda i: (i, 0))],
        out_shape=[jax.ShapeDtypeStruct((ENT2_ROWS, DP), jnp.float32),
                   jax.ShapeDtypeStruct((REL2_ROWS, DP), jnp.float32)],
        compiler_params=pltpu.CompilerParams(
            dimension_semantics=("parallel",)),
    )(et, rnt, rht)


def _make_sc_gather(n_sets, table_rows):
    mesh = plsc.VectorSubcoreMesh(core_axis_name="c", subcore_axis_name="s")
    row_t = jax.ShapeDtypeStruct((B, DP), jnp.float32)
    n_items = n_sets * (PER_W // CHUNK)

    @functools.partial(
        pl.kernel,
        mesh=mesh,
        out_type=[row_t] * n_sets,
        scratch_types=[
            pltpu.VMEM((n_sets * PER_W,), jnp.int32),
            pltpu.VMEM((CHUNK, DP), jnp.float32),
            pltpu.VMEM((CHUNK, DP), jnp.float32),
            pltpu.SemaphoreType.DMA,
            pltpu.SemaphoreType.DMA,
            pltpu.SemaphoreType.DMA,
        ],
        compiler_params=pltpu.CompilerParams(use_tc_tiling_on_sc=True),
    )
    def k(*refs):
        idx_hbms = refs[:n_sets]
        tab_hbm = refs[n_sets]
        outs = refs[n_sets + 1:2 * n_sets + 1]
        idx_v, buf0, buf1, gsem0, gsem1, ssem = refs[2 * n_sets + 1:]
        wid = lax.axis_index("s") * NC + lax.axis_index("c")
        base = wid * PER_W
        bufs = (buf0, buf1)
        gsems = (gsem0, gsem1)

        icopies = [
            pltpu.async_copy(src.at[pl.ds(base, PER_W)],
                             idx_v.at[pl.ds(s * PER_W, PER_W)], ssem)
            for s, src in enumerate(idx_hbms)
        ]
        for cp in icopies:
            cp.wait()

        def fire(item, buf, gsem):
            st, chunk = divmod(item, PER_W // CHUNK)
            cps = []
            for c in range(CHUNK // IDX_CHUNK):
                off = st * PER_W + chunk * CHUNK + c * IDX_CHUNK
                cps.append(pltpu.async_copy(
                    tab_hbm.at[idx_v.at[pl.ds(off, IDX_CHUNK)]],
                    buf.at[pl.ds(c * IDX_CHUNK, IDX_CHUNK)],
                    gsem,
                ))
            return cps

        def store(item, buf):
            st, chunk = divmod(item, PER_W // CHUNK)
            return pltpu.async_copy(
                buf, outs[st].at[pl.ds(base + chunk * CHUNK, CHUNK)], ssem)

        gathers = [None] * n_items
        stores = [None] * n_items
        gathers[0] = fire(0, bufs[0], gsems[0])
        for item in range(n_items):
            par = item % 2
            for cp in gathers[item]:
                cp.wait()
            if item + 1 < n_items:
                if item >= 1:
                    stores[item - 1].wait()
                gathers[item + 1] = fire(item + 1, bufs[1 - par],
                                         gsems[1 - par])
            stores[item] = store(item, bufs[par])
        stores[n_items - 1].wait()
        if n_items >= 2:
            stores[n_items - 2].wait()

    return k


def _half(pair_block, sel_col):
    return jnp.where(sel_col == 0.0, pair_block[:, :D], pair_block[:, D:])


def _groupsum(parts):
    k = len(parts)
    x = jnp.concatenate(parts, axis=1)
    g = jax.lax.broadcasted_iota(jnp.int32, (k * D, k), 0) // D == \
        jax.lax.broadcasted_iota(jnp.int32, (k * D, k), 1)
    return jax.lax.dot_general(
        x, g.astype(jnp.float32), (((1,), (0,)), ((), ())),
        preferred_element_type=jnp.float32)


def _tc_loss_body(hp_r, tp_r, chp_r, ctp_r, rel_r, sel_r, out_r):
    sel = sel_r[...]
    hd = _half(hp_r[...], sel[:, 0:1])
    tl = _half(tp_r[...], sel[:, 1:2])
    c_h = _half(chp_r[...], sel[:, 2:3])
    c_t = _half(ctp_r[...], sel[:, 3:4])
    rel = rel_r[...]
    rn = rel[:, :D]
    rh = rel[:, D:]

    d = hd - tl
    dc = c_h - c_t
    s = _groupsum([rn * d, rn * dc])
    pv = d - s[:, 0:1] * rn + rh + 1e-6
    nv = dc - s[:, 1:2] * rn + rh + 1e-6
    q = _groupsum([pv * pv, nv * nv, hd * hd, tl * tl, c_h * c_h, c_t * c_t])
    pos = jnp.sqrt(q[:, 0:1])
    neg = jnp.sqrt(q[:, 1:2])
    total = jnp.sum(jnp.maximum(pos - neg + 1.0, 0.0))
    nrm = jnp.sqrt(q[:, 2:6])
    total += jnp.sum(jnp.maximum(nrm - 1.0, 0.0))
    out_r[0, 0, 0] = total


def _tc_loss(headp, tailp, chp, ctp, rel, sels):
    out = pl.pallas_call(
        _tc_loss_body,
        grid=(NB,),
        in_specs=[pl.BlockSpec((TC_BLK, DP), lambda i: (i, 0))] * 5
        + [pl.BlockSpec((TC_BLK, 4), lambda i: (i, 0))],
        out_specs=pl.BlockSpec((1, 1, 1), lambda i: (i, 0, 0),
                               memory_space=pltpu.SMEM),
        out_shape=jax.ShapeDtypeStruct((NB, 1, 1), jnp.float32),
        compiler_params=pltpu.CompilerParams(
            dimension_semantics=("parallel",)),
    )(headp, tailp, chp, ctp, rel, sels)
    return jnp.sum(out)


def kernel(current_triples, corrupted_triples, entity_embedding,
           relation_norm_embedding, relation_hyper_embedding):
    h = current_triples[:, 0]
    t = current_triples[:, 1]
    r = current_triples[:, 2]
    h_c = corrupted_triples[:, 0]
    t_c = corrupted_triples[:, 1]

    ent2, rel2 = _prep(entity_embedding.T, relation_norm_embedding.T,
                       relation_hyper_embedding.T)

    def fold(e):
        return ((e >> 10) << 9) + (e & 511)

    gather_ent = _make_sc_gather(4, ENT2_ROWS)
    gather_rel = _make_sc_gather(1, REL2_ROWS)
    headp, tailp, chp, ctp = gather_ent(
        fold(h), fold(t), fold(h_c), fold(t_c), ent2)
    (rel,) = gather_rel(r, rel2)

    sels = jnp.stack(
        [(h >> 9) & 1, (t >> 9) & 1, (h_c >> 9) & 1, (t_c >> 9) & 1],
        axis=1).astype(jnp.float32)
    return _tc_loss(headp, tailp, chp, ctp, rel, sels)
